# baseline (speedup 1.0000x reference)
import functools

import numpy as np
import jax
import jax.numpy as jnp
from jax import lax
from jax.experimental import pallas as pl
from jax.experimental.pallas import tpu as pltpu
from jax.experimental.pallas import tpu_sc as plsc

N_E = 1024
E_DIM = 256
B_TOK = 9216
SK_EPSILON = 0.003

TILE = 256
NT = B_TOK // TILE
N_CHEAP = 24
N_POLISH = 6

_SPLITC = np.float32(4097.0)


def _two_sum(a, b):
    s = a + b
    bb = s - a
    err = (a - (s - bb)) + (b - bb)
    return s, err


def _quick_two_sum(a, b):
    s = a + b
    err = b - (s - a)
    return s, err


def _two_prod(a, b):
    p = a * b
    t = a * _SPLITC
    ah = t - (t - a)
    al = a - ah
    t2 = b * _SPLITC
    bh = t2 - (t2 - b)
    bl = b - bh
    err = ((ah * bh - p) + ah * bl + al * bh) + al * bl
    return p, err


def _ds_add(ah, al, bh, bl):
    sh, se = _two_sum(ah, bh)
    se = se + (al + bl)
    return _quick_two_sum(sh, se)


def _ds_add_f32(ah, al, b):
    sh, se = _two_sum(ah, b)
    se = se + al
    return _quick_two_sum(sh, se)


def _ds_mul(ah, al, bh, bl):
    ph, pe = _two_prod(ah, bh)
    pe = pe + (ah * bl + al * bh)
    return _quick_two_sum(ph, pe)


def _ds_mul_c(a, ch, cl):
    ph, pe = _two_prod(a, ch)
    pe = pe + a * cl
    return _quick_two_sum(ph, pe)


_LOG2E = np.float32(1.4426950408889634)
_LN2 = 0.6931471805599453094172321
_L2HI = np.float32(0.693145751953125)
_L2MID = np.float32(_LN2 - float(_L2HI))
_L2LO = np.float32(_LN2 - float(_L2HI) - float(_L2MID))
_SIXTH_H = np.float32(1.0 / 6.0)
_SIXTH_L = np.float32(1.0 / 6.0 - float(np.float32(1.0 / 6.0)))
_TC = [np.float32(1.0 / 3628800.0), np.float32(1.0 / 362880.0),
       np.float32(1.0 / 40320.0), np.float32(1.0 / 5040.0),
       np.float32(1.0 / 720.0), np.float32(1.0 / 120.0),
       np.float32(1.0 / 24.0)]


def _exp_ds(uh, ul):
    mask = uh > np.float32(-60.0)
    uc = jnp.maximum(uh, np.float32(-100.0))
    n = jnp.round(uc * _LOG2E)
    rh = uc - n * _L2HI
    t1h, t1l = _two_sum(rh, -(n * _L2MID))
    rl = t1l + (ul - n * _L2LO)
    rh2, rl2 = _quick_two_sum(t1h, rl)
    r2h, r2l = _two_prod(rh2, rh2)
    r2l = r2l + np.float32(2.0) * rh2 * rl2
    h2h = r2h * np.float32(0.5)
    h2l = r2l * np.float32(0.5)
    r3h, r3e = _two_prod(r2h, rh2)
    r3e = r3e + r2l * rh2
    r3h, r3e = _quick_two_sum(r3h, r3e)
    t3h, t3e = _two_prod(r3h, _SIXTH_H)
    t3e = t3e + (r3h * _SIXTH_L + r3e * _SIXTH_H)
    t3h, t3e = _quick_two_sum(t3h, t3e)
    w = _TC[0]
    for c in _TC[1:]:
        w = w * rh2 + c
    r2f = rh2 * rh2
    w = (r2f * r2f) * w
    ah, al = _two_sum(np.float32(1.0), rh2)
    al = al + rl2
    ah, al = _quick_two_sum(ah, al)
    ah, al = _ds_add(ah, al, h2h, h2l)
    ah, al = _ds_add(ah, al, t3h, t3e)
    ah, al = _ds_add_f32(ah, al, w)
    nbits = lax.shift_left(n.astype(jnp.int32) + 127, 23)
    scale = lax.bitcast_convert_type(nbits, jnp.float32)
    zero = jnp.zeros_like(ah)
    eh = jnp.where(mask, ah * scale, zero)
    el = jnp.where(mask, al * scale, zero)
    return eh, el


def _log_ds(sh, sl):
    y0 = jnp.log(sh)
    eh, el = _exp_ds(-y0, jnp.zeros_like(y0))
    zh, zl = _ds_mul(sh, sl, eh, el)
    dh, dl = _ds_add_f32(zh, zl, np.float32(-1.0))
    lh, ll = _ds_add_f32(dh, dl, -(dh * dh * np.float32(0.5)))
    th, te = _two_sum(y0, lh)
    te = te + ll
    return _quick_two_sum(th, te)


def _ds_fold1(eh, el):
    w = eh.shape[1]
    while w > 1:
        h = w // 2
        eh, el = _ds_add(eh[:, :h], el[:, :h], eh[:, h:w], el[:, h:w])
        w = h
    return eh, el


def _ds_fold0(eh, el):
    r = eh.shape[0]
    while r > 1:
        h = r // 2
        eh, el = _ds_add(eh[:h, :], el[:h, :], eh[h:r, :], el[h:r, :])
        r = h
    return eh, el


def _dist_body(lat_ref, a_ref, cb_ref, b_ref, d_ref):
    m = lax.dot_general(lat_ref[...], cb_ref[...],
                        dimension_numbers=(((1,), (1,)), ((), ())),
                        precision=lax.Precision.DEFAULT,
                        preferred_element_type=jnp.float32)
    d_ref[...] = (a_ref[...] + b_ref[...]) - np.float32(2.0) * m


_CF64 = -1.0 / SK_EPSILON
_CF_H = np.float32(_CF64)
_CF_L = np.float32(_CF64 - float(_CF_H))
_NEG_BIG = np.float32(-3.0e38)


def _sinkhorn_body(d_hbm_ref, idx_ref, loss_ref, dc_ref, dma_sem):
    cp = pltpu.make_async_copy(d_hbm_ref, dc_ref, dma_sem)
    cp.start()
    cp.wait()

    def mm_tile(i, carry):
        mx, mn = carry
        t = dc_ref[pl.ds(i * TILE, TILE), :]
        return jnp.maximum(mx, jnp.max(t)), jnp.minimum(mn, jnp.min(t))

    mx0 = jnp.full((), _NEG_BIG, jnp.float32)
    mn0 = jnp.full((), -_NEG_BIG, jnp.float32)
    mx, mn = lax.fori_loop(0, NT, mm_tile, (mx0, mn0))
    mid = (mx + mn) / np.float32(2.0)
    amp = mx - mid + np.float32(1e-5)

    dcmin = (mn - mid) / amp
    ovf = dcmin < np.float32(-0.2662)
    nt_dyn = jnp.where(ovf, 0, NT)
    nt_ovf = jnp.where(ovf, NT, 0)
    ncheap_dyn = jnp.where(ovf, 0, N_CHEAP)
    npol_dyn = jnp.where(ovf, 0, N_POLISH)

    def center_tile(i, c):
        sl = pl.ds(i * TILE, TILE)
        dc_ref[sl, :] = (dc_ref[sl, :] - mid) / amp
        return c

    lax.fori_loop(0, nt_dyn, center_tile, jnp.float32(0.0))

    def ovf_tile(i, acc):
        sl = pl.ds(i * TILE, TILE)
        blk = (dc_ref[sl, 0:128] - mid) / amp
        lane0 = lax.broadcasted_iota(jnp.int32, (TILE, 128), 1) == 0
        idx_ref[sl, :] = jnp.zeros((TILE, 1), jnp.int32)
        return acc + jnp.sum(jnp.where(lane0, blk, np.float32(0.0)))

    d0sum = lax.fori_loop(0, nt_ovf, ovf_tile, jnp.float32(0.0))

    def col0_tile(i, carry):
        cm, cs = carry
        s = dc_ref[pl.ds(i * TILE, TILE), :] * _CF_H
        tm = jnp.max(s, axis=0, keepdims=True)
        nm = jnp.maximum(cm, tm)
        cs = cs * jnp.exp(cm - nm) + jnp.sum(jnp.exp(s - nm), axis=0,
                                             keepdims=True)
        return nm, cs

    cm0 = jnp.full((1, N_E), _NEG_BIG, jnp.float32)
    cs0 = jnp.zeros((1, N_E), jnp.float32)
    cm, cs = lax.fori_loop(0, nt_dyn, col0_tile, (cm0, cs0))
    gamma0 = -(jnp.log(cs) + cm)

    def cheap_pass(_, gamma):
        def tile(i, carry):
            cm, cs = carry
            s = dc_ref[pl.ds(i * TILE, TILE), :] * _CF_H
            t2 = s + gamma
            m2 = jnp.max(t2, axis=1, keepdims=True)
            rs = jnp.sum(jnp.exp(t2 - m2), axis=1, keepdims=True)
            phi = -(jnp.log(rs) + m2)
            t = s + phi
            tm = jnp.max(t, axis=0, keepdims=True)
            nm = jnp.maximum(cm, tm)
            cs = cs * jnp.exp(cm - nm) + jnp.sum(jnp.exp(t - nm), axis=0,
                                                 keepdims=True)
            return nm, cs

        cm, cs = lax.fori_loop(0, NT, tile, (cm0, cs0))
        return -(jnp.log(cs) + cm)

    gamma1 = lax.fori_loop(0, ncheap_dyn, cheap_pass, gamma0)

    def polish_pass(_, g):
        gh, gl = g

        def tile(i, carry):
            cm, csh, csl = carry
            dc = dc_ref[pl.ds(i * TILE, TILE), :]
            sh, slo = _ds_mul_c(dc, _CF_H, _CF_L)
            th, tl = _ds_add(sh, slo, gh, gl)
            m2 = jnp.max(th, axis=1, keepdims=True)
            uh, ue = _two_sum(th, -m2)
            ue = ue + tl
            uh, ul = _quick_two_sum(uh, ue)
            eh, el = _exp_ds(uh, ul)
            rsh, rsl = _ds_fold1(eh, el)
            lh, ll = _log_ds(rsh, rsl)
            ph, pe = _two_sum(lh, m2)
            pe = pe + ll
            ph, pl_ = _quick_two_sum(ph, pe)
            phih, phil = -ph, -pl_
            t2h, t2l = _ds_add(sh, slo, phih, phil)
            tm = jnp.max(t2h, axis=0, keepdims=True)
            nm = jnp.maximum(cm, tm)
            dh, de = _two_sum(cm, -nm)
            rh_, rl_ = _exp_ds(dh, de)
            csh, csl = _ds_mul(csh, csl, rh_, rl_)
            vh, ve = _two_sum(t2h, -nm)
            ve = ve + t2l
            vh, vl = _quick_two_sum(vh, ve)
            eh2, el2 = _exp_ds(vh, vl)
            tsh, tsl = _ds_fold0(eh2, el2)
            csh, csl = _ds_add(csh, csl, tsh, tsl)
            return nm, csh, csl

        csh0 = jnp.zeros((1, N_E), jnp.float32)
        cm, csh, csl = lax.fori_loop(0, NT, tile, (cm0, csh0, csh0))
        lh, ll = _log_ds(csh, csl)
        gh2, ge = _two_sum(lh, cm)
        ge = ge + ll
        gh2, gl2 = _quick_two_sum(gh2, ge)
        return -gh2, -gl2

    gh, gl = lax.fori_loop(0, npol_dyn, polish_pass,
                           (gamma1, jnp.zeros((1, N_E), jnp.float32)))

    def arg_tile(i, acc):
        dc = dc_ref[pl.ds(i * TILE, TILE), :]
        sh, slo = _ds_mul_c(dc, _CF_H, _CF_L)
        th, tl = _ds_add(sh, slo, gh, gl)
        m2 = jnp.max(th, axis=1, keepdims=True)
        zh, ze = _two_sum(th, -m2)
        zf = zh + (ze + tl)
        best = jnp.max(zf, axis=1, keepdims=True)
        ii = lax.broadcasted_iota(jnp.int32, (TILE, N_E), 1)
        sel = jnp.where(zf == best, ii, jnp.int32(2 ** 30))
        idx = jnp.min(sel, axis=1, keepdims=True)
        idx_ref[pl.ds(i * TILE, TILE), :] = idx
        pick = jnp.sum(jnp.where(ii == idx, dc, np.float32(0.0)),
                       axis=1, keepdims=True)
        return acc + jnp.sum(pick)

    dcsum = lax.fori_loop(0, nt_dyn, arg_tile, jnp.float32(0.0))
    dcsum = dcsum + d0sum
    total = amp * dcsum + mid * np.float32(B_TOK)
    loss = total / np.float32(B_TOK * E_DIM) * np.float32(1.25)
    loss_ref[...] = loss * jnp.ones((1, 1), jnp.float32)


def kernel(x, codebook):
    orig_shape = x.shape
    with jax.enable_x64(False):
        lat = x.reshape(-1, E_DIM)
        a = jnp.sum(lat ** 2, axis=1, keepdims=True)
        b = jnp.sum(codebook ** 2, axis=1)[None, :]

        d = pl.pallas_call(
            _dist_body,
            grid=(NT,),
            in_specs=[
                pl.BlockSpec((TILE, E_DIM), lambda i: (i, i * 0)),
                pl.BlockSpec((TILE, 1), lambda i: (i, i * 0)),
                pl.BlockSpec((N_E, E_DIM), lambda i: (i * 0, i * 0)),
                pl.BlockSpec((1, N_E), lambda i: (i * 0, i * 0)),
            ],
            out_specs=pl.BlockSpec((TILE, N_E), lambda i: (i, i * 0)),
            out_shape=jax.ShapeDtypeStruct((B_TOK, N_E), jnp.float32),
        )(lat, a, codebook, b)

        idx, loss = pl.pallas_call(
            _sinkhorn_body,
            in_specs=[pl.BlockSpec(memory_space=pl.ANY)],
            out_shape=(jax.ShapeDtypeStruct((B_TOK, 1), jnp.int32),
                       jax.ShapeDtypeStruct((1, 1), jnp.float32)),
            scratch_shapes=[pltpu.VMEM((B_TOK, N_E), jnp.float32),
                            pltpu.SemaphoreType.DMA],
        )(d)
        idx_flat = idx.reshape(B_TOK)

        info = plsc.get_sparse_core_info()
        nw = info.num_cores * info.num_subcores
        bpw = B_TOK // nw
        mesh = plsc.VectorSubcoreMesh(core_axis_name="c", subcore_axis_name="s")

        nchunk = 4
        ch = bpw // nchunk

        @functools.partial(
            pl.kernel, mesh=mesh,
            out_type=jax.ShapeDtypeStruct((B_TOK, E_DIM), jnp.float32),
            scratch_types=[
                pltpu.VMEM((bpw,), jnp.int32),
                pltpu.VMEM((bpw, E_DIM), jnp.float32),
                pltpu.SemaphoreType.DMA,
                pltpu.SemaphoreType.DMA,
            ],
        )
        def _sc_gather(table_hbm, idx_hbm, out_hbm, idx_v, rows_v, gsem, osem):
            wid = lax.axis_index("s") * info.num_cores + lax.axis_index("c")
            base = wid * bpw
            pltpu.sync_copy(idx_hbm.at[pl.ds(base, bpw)], idx_v)
            gathers = [
                pltpu.async_copy(
                    table_hbm.at[idx_v.at[pl.ds(c * ch, ch)]],
                    rows_v.at[pl.ds(c * ch, ch)], gsem)
                for c in range(nchunk)
            ]
            outs = []
            for c in range(nchunk):
                gathers[c].wait()
                outs.append(pltpu.async_copy(
                    rows_v.at[pl.ds(c * ch, ch)],
                    out_hbm.at[pl.ds(base + c * ch, ch)], osem))
            for o in outs:
                o.wait()

        x_q = _sc_gather(codebook, idx_flat).reshape(orig_shape)
        x_q_st = x + (x_q - x)
        loss_s = loss[0, 0]
    indices = idx_flat.reshape(orig_shape[:-1]).astype(jnp.int64)
    return (x_q_st, loss_s, indices)


# speedup vs baseline: 1.0112x; 1.0112x over previous
import functools

import numpy as np
import jax
import jax.numpy as jnp
from jax import lax
from jax.experimental import pallas as pl
from jax.experimental.pallas import tpu as pltpu
from jax.experimental.pallas import tpu_sc as plsc

N_E = 1024
E_DIM = 256
B_TOK = 9216
SK_EPSILON = 0.003

TILE = 256
NT = B_TOK // TILE
N_CHEAP = 24
N_POLISH = 6

_SPLITC = np.float32(4097.0)


def _two_sum(a, b):
    s = a + b
    bb = s - a
    err = (a - (s - bb)) + (b - bb)
    return s, err


def _quick_two_sum(a, b):
    s = a + b
    err = b - (s - a)
    return s, err


def _two_prod(a, b):
    p = a * b
    t = a * _SPLITC
    ah = t - (t - a)
    al = a - ah
    t2 = b * _SPLITC
    bh = t2 - (t2 - b)
    bl = b - bh
    err = ((ah * bh - p) + ah * bl + al * bh) + al * bl
    return p, err


def _ds_add(ah, al, bh, bl):
    sh, se = _two_sum(ah, bh)
    se = se + (al + bl)
    return _quick_two_sum(sh, se)


def _ds_add_f32(ah, al, b):
    sh, se = _two_sum(ah, b)
    se = se + al
    return _quick_two_sum(sh, se)


def _ds_mul(ah, al, bh, bl):
    ph, pe = _two_prod(ah, bh)
    pe = pe + (ah * bl + al * bh)
    return _quick_two_sum(ph, pe)


def _ds_mul_c(a, ch, cl):
    ph, pe = _two_prod(a, ch)
    pe = pe + a * cl
    return _quick_two_sum(ph, pe)


_LOG2E = np.float32(1.4426950408889634)
_LN2 = 0.6931471805599453094172321
_L2HI = np.float32(0.693145751953125)
_L2MID = np.float32(_LN2 - float(_L2HI))
_L2LO = np.float32(_LN2 - float(_L2HI) - float(_L2MID))
_SIXTH_H = np.float32(1.0 / 6.0)
_SIXTH_L = np.float32(1.0 / 6.0 - float(np.float32(1.0 / 6.0)))
_TC = [np.float32(1.0 / 3628800.0), np.float32(1.0 / 362880.0),
       np.float32(1.0 / 40320.0), np.float32(1.0 / 5040.0),
       np.float32(1.0 / 720.0), np.float32(1.0 / 120.0),
       np.float32(1.0 / 24.0)]


def _exp_ds(uh, ul):
    mask = uh > np.float32(-60.0)
    uc = jnp.maximum(uh, np.float32(-100.0))
    n = jnp.round(uc * _LOG2E)
    rh = uc - n * _L2HI
    t1h, t1l = _two_sum(rh, -(n * _L2MID))
    rl = t1l + (ul - n * _L2LO)
    rh2, rl2 = _quick_two_sum(t1h, rl)
    r2h, r2l = _two_prod(rh2, rh2)
    r2l = r2l + np.float32(2.0) * rh2 * rl2
    h2h = r2h * np.float32(0.5)
    h2l = r2l * np.float32(0.5)
    r3h, r3e = _two_prod(r2h, rh2)
    r3e = r3e + r2l * rh2
    r3h, r3e = _quick_two_sum(r3h, r3e)
    t3h, t3e = _two_prod(r3h, _SIXTH_H)
    t3e = t3e + (r3h * _SIXTH_L + r3e * _SIXTH_H)
    t3h, t3e = _quick_two_sum(t3h, t3e)
    w = _TC[0]
    for c in _TC[1:]:
        w = w * rh2 + c
    r2f = rh2 * rh2
    w = (r2f * r2f) * w
    ah, al = _two_sum(np.float32(1.0), rh2)
    al = al + rl2
    ah, al = _quick_two_sum(ah, al)
    ah, al = _ds_add(ah, al, h2h, h2l)
    ah, al = _ds_add(ah, al, t3h, t3e)
    ah, al = _ds_add_f32(ah, al, w)
    nbits = lax.shift_left(n.astype(jnp.int32) + 127, 23)
    scale = lax.bitcast_convert_type(nbits, jnp.float32)
    zero = jnp.zeros_like(ah)
    eh = jnp.where(mask, ah * scale, zero)
    el = jnp.where(mask, al * scale, zero)
    return eh, el


def _log_ds(sh, sl):
    y0 = jnp.log(sh)
    eh, el = _exp_ds(-y0, jnp.zeros_like(y0))
    zh, zl = _ds_mul(sh, sl, eh, el)
    dh, dl = _ds_add_f32(zh, zl, np.float32(-1.0))
    lh, ll = _ds_add_f32(dh, dl, -(dh * dh * np.float32(0.5)))
    th, te = _two_sum(y0, lh)
    te = te + ll
    return _quick_two_sum(th, te)


def _ds_fold1(eh, el):
    w = eh.shape[1]
    while w > 1:
        h = w // 2
        eh, el = _ds_add(eh[:, :h], el[:, :h], eh[:, h:w], el[:, h:w])
        w = h
    return eh, el


def _ds_fold0(eh, el):
    r = eh.shape[0]
    while r > 1:
        h = r // 2
        eh, el = _ds_add(eh[:h, :], el[:h, :], eh[h:r, :], el[h:r, :])
        r = h
    return eh, el


def _dist_body(lat_ref, a_ref, cb_ref, b_ref, d_ref):
    m = lax.dot_general(lat_ref[...], cb_ref[...],
                        dimension_numbers=(((1,), (1,)), ((), ())),
                        precision=lax.Precision.DEFAULT,
                        preferred_element_type=jnp.float32)
    d_ref[...] = (a_ref[...] + b_ref[...]) - np.float32(2.0) * m


_CF64 = -1.0 / SK_EPSILON
_CF_H = np.float32(_CF64)
_CF_L = np.float32(_CF64 - float(_CF_H))
_NEG_BIG = np.float32(-3.0e38)


def _sinkhorn_body(d_hbm_ref, idx_ref, loss_ref, dc_ref, dma_sem):
    cp = pltpu.make_async_copy(d_hbm_ref, dc_ref, dma_sem)
    cp.start()
    cp.wait()

    def mm_tile(i, carry):
        mx, mn = carry
        t = dc_ref[pl.ds(i * TILE, TILE), :]
        return jnp.maximum(mx, jnp.max(t)), jnp.minimum(mn, jnp.min(t))

    mx0 = jnp.full((), _NEG_BIG, jnp.float32)
    mn0 = jnp.full((), -_NEG_BIG, jnp.float32)
    mx, mn = lax.fori_loop(0, NT, mm_tile, (mx0, mn0))
    mid = (mx + mn) / np.float32(2.0)
    amp = mx - mid + np.float32(1e-5)

    dcmin = (mn - mid) / amp
    ovf = dcmin < np.float32(-0.2662)
    nt_dyn = jnp.where(ovf, 0, NT)
    nt_ovf = jnp.where(ovf, NT, 0)
    ncheap_dyn = jnp.where(ovf, 0, N_CHEAP)
    npol_dyn = jnp.where(ovf, 0, N_POLISH)

    def center_tile(i, c):
        sl = pl.ds(i * TILE, TILE)
        dc_ref[sl, :] = (dc_ref[sl, :] - mid) / amp
        return c

    lax.fori_loop(0, nt_dyn, center_tile, jnp.float32(0.0))

    def ovf_tile(i, acc):
        sl = pl.ds(i * TILE, TILE)
        blk = (dc_ref[sl, 0:128] - mid) / amp
        lane0 = lax.broadcasted_iota(jnp.int32, (TILE, 128), 1) == 0
        idx_ref[sl, :] = jnp.zeros((TILE, 1), jnp.int32)
        return acc + jnp.sum(jnp.where(lane0, blk, np.float32(0.0)))

    d0sum = lax.fori_loop(0, nt_ovf, ovf_tile, jnp.float32(0.0))

    def col0_tile(i, carry):
        cm, cs = carry
        s = dc_ref[pl.ds(i * TILE, TILE), :] * _CF_H
        tm = jnp.max(s, axis=0, keepdims=True)
        nm = jnp.maximum(cm, tm)
        cs = cs * jnp.exp(cm - nm) + jnp.sum(jnp.exp(s - nm), axis=0,
                                             keepdims=True)
        return nm, cs

    cm0 = jnp.full((1, N_E), _NEG_BIG, jnp.float32)
    cs0 = jnp.zeros((1, N_E), jnp.float32)
    cm, cs = lax.fori_loop(0, nt_dyn, col0_tile, (cm0, cs0))
    gamma0 = -(jnp.log(cs) + cm)

    def cheap_pass(_, gamma):
        def tile(i, carry):
            cm, cs = carry
            s = dc_ref[pl.ds(i * TILE, TILE), :] * _CF_H
            t2 = s + gamma
            m2 = jnp.max(t2, axis=1, keepdims=True)
            rs = jnp.sum(jnp.exp(t2 - m2), axis=1, keepdims=True)
            phi = -(jnp.log(rs) + m2)
            t = s + phi
            tm = jnp.max(t, axis=0, keepdims=True)
            nm = jnp.maximum(cm, tm)
            cs = cs * jnp.exp(cm - nm) + jnp.sum(jnp.exp(t - nm), axis=0,
                                                 keepdims=True)
            return nm, cs

        cm, cs = lax.fori_loop(0, NT, tile, (cm0, cs0))
        return -(jnp.log(cs) + cm)

    gamma1 = lax.fori_loop(0, ncheap_dyn, cheap_pass, gamma0)

    def polish_pass(_, g):
        gh, gl = g

        def tile(i, carry):
            cm, csh, csl = carry
            dc = dc_ref[pl.ds(i * TILE, TILE), :]
            sh, slo = _ds_mul_c(dc, _CF_H, _CF_L)
            th, tl = _ds_add(sh, slo, gh, gl)
            m2 = jnp.max(th, axis=1, keepdims=True)
            uh, ue = _two_sum(th, -m2)
            ue = ue + tl
            uh, ul = _quick_two_sum(uh, ue)
            eh, el = _exp_ds(uh, ul)
            rsh, rsl = _ds_fold1(eh, el)
            lh, ll = _log_ds(rsh, rsl)
            ph, pe = _two_sum(lh, m2)
            pe = pe + ll
            ph, pl_ = _quick_two_sum(ph, pe)
            phih, phil = -ph, -pl_
            t2h, t2l = _ds_add(sh, slo, phih, phil)
            tm = jnp.max(t2h, axis=0, keepdims=True)
            nm = jnp.maximum(cm, tm)
            dh, de = _two_sum(cm, -nm)
            rh_, rl_ = _exp_ds(dh, de)
            csh, csl = _ds_mul(csh, csl, rh_, rl_)
            vh, ve = _two_sum(t2h, -nm)
            ve = ve + t2l
            vh, vl = _quick_two_sum(vh, ve)
            eh2, el2 = _exp_ds(vh, vl)
            tsh, tsl = _ds_fold0(eh2, el2)
            csh, csl = _ds_add(csh, csl, tsh, tsl)
            return nm, csh, csl

        csh0 = jnp.zeros((1, N_E), jnp.float32)
        cm, csh, csl = lax.fori_loop(0, NT, tile, (cm0, csh0, csh0))
        lh, ll = _log_ds(csh, csl)
        gh2, ge = _two_sum(lh, cm)
        ge = ge + ll
        gh2, gl2 = _quick_two_sum(gh2, ge)
        return -gh2, -gl2

    gh, gl = lax.fori_loop(0, npol_dyn, polish_pass,
                           (gamma1, jnp.zeros((1, N_E), jnp.float32)))

    def arg_tile(i, acc):
        dc = dc_ref[pl.ds(i * TILE, TILE), :]
        sh, slo = _ds_mul_c(dc, _CF_H, _CF_L)
        th, tl = _ds_add(sh, slo, gh, gl)
        m2 = jnp.max(th, axis=1, keepdims=True)
        zh, ze = _two_sum(th, -m2)
        zf = zh + (ze + tl)
        best = jnp.max(zf, axis=1, keepdims=True)
        ii = lax.broadcasted_iota(jnp.int32, (TILE, N_E), 1)
        sel = jnp.where(zf == best, ii, jnp.int32(2 ** 30))
        idx = jnp.min(sel, axis=1, keepdims=True)
        idx_ref[pl.ds(i * TILE, TILE), :] = idx
        pick = jnp.sum(jnp.where(ii == idx, dc, np.float32(0.0)),
                       axis=1, keepdims=True)
        return acc + jnp.sum(pick)

    dcsum = lax.fori_loop(0, nt_dyn, arg_tile, jnp.float32(0.0))
    dcsum = dcsum + d0sum
    total = amp * dcsum + mid * np.float32(B_TOK)
    loss = total / np.float32(B_TOK * E_DIM) * np.float32(1.25)
    loss_ref[...] = loss * jnp.ones((1, 1), jnp.float32)


def kernel(x, codebook):
    orig_shape = x.shape
    with jax.enable_x64(False):
        lat = x.reshape(-1, E_DIM)
        a = jnp.sum(lat ** 2, axis=1, keepdims=True)
        b = jnp.sum(codebook ** 2, axis=1)[None, :]

        d = pl.pallas_call(
            _dist_body,
            grid=(NT,),
            in_specs=[
                pl.BlockSpec((TILE, E_DIM), lambda i: (i, i * 0)),
                pl.BlockSpec((TILE, 1), lambda i: (i, i * 0)),
                pl.BlockSpec((N_E, E_DIM), lambda i: (i * 0, i * 0)),
                pl.BlockSpec((1, N_E), lambda i: (i * 0, i * 0)),
            ],
            out_specs=pl.BlockSpec((TILE, N_E), lambda i: (i, i * 0)),
            out_shape=jax.ShapeDtypeStruct((B_TOK, N_E), jnp.float32),
        )(lat, a, codebook, b)

        idx, loss = pl.pallas_call(
            _sinkhorn_body,
            in_specs=[pl.BlockSpec(memory_space=pl.ANY)],
            out_shape=(jax.ShapeDtypeStruct((B_TOK, 1), jnp.int32),
                       jax.ShapeDtypeStruct((1, 1), jnp.float32)),
            scratch_shapes=[pltpu.VMEM((B_TOK, N_E), jnp.float32),
                            pltpu.SemaphoreType.DMA],
        )(d)
        idx_flat = idx.reshape(B_TOK)

        info = plsc.get_sparse_core_info()
        nw = info.num_cores * info.num_subcores
        bpw = B_TOK // nw
        mesh = plsc.VectorSubcoreMesh(core_axis_name="c", subcore_axis_name="s")

        @functools.partial(
            pl.kernel, mesh=mesh,
            out_type=jax.ShapeDtypeStruct((B_TOK, E_DIM), jnp.float32),
            scratch_types=[
                pltpu.VMEM((bpw,), jnp.int32),
                pltpu.VMEM((bpw, E_DIM), jnp.float32),
                pltpu.SemaphoreType.DMA,
            ],
        )
        def _sc_gather(table_hbm, idx_hbm, out_hbm, idx_v, rows_v, sem):
            wid = lax.axis_index("s") * info.num_cores + lax.axis_index("c")
            base = wid * bpw
            pltpu.sync_copy(idx_hbm.at[pl.ds(base, bpw)], idx_v)
            pltpu.async_copy(table_hbm.at[idx_v], rows_v, sem).wait()
            pltpu.sync_copy(rows_v, out_hbm.at[pl.ds(base, bpw)])

        x_q = _sc_gather(codebook, idx_flat).reshape(orig_shape)
        x_q_st = x + (x_q - x)
        loss_s = loss[0, 0]
    indices = idx_flat.reshape(orig_shape[:-1]).astype(jnp.int64)
    return (x_q_st, loss_s, indices)
